# Initial kernel scaffold; baseline (speedup 1.0000x reference)
import jax
import jax.numpy as jnp
from jax.experimental import pallas as pl


def kernel(title_ids, token_ids, title_table, text_table):
    raise NotImplementedError("write your pallas kernel here")


# speedup vs baseline: 12.5722x; 12.5722x over previous
import functools

import jax
import jax.numpy as jnp
from jax import lax
from jax.experimental import pallas as pl
from jax.experimental.pallas import tpu as pltpu
from jax.experimental.pallas import tpu_sc as plsc

B = 16384
L = 20
D = 32
TITLE_V = 100001
TEXT_V = 10000

NC = 2
NS = 16
NW = NC * NS
BPW = B // NW
CHUNK = 32
NCHUNK = BPW // CHUNK
ROWS_PER_CHUNK = CHUNK * L
IDX_ROWS_PER_CHUNK = ROWS_PER_CHUNK // 128


def _body(title_idx_h, tok_bmaj_h, tok_tmaj_h, title_table_h, text_table_h,
          out_h, tokb_v, tokt_v, tidx_v, trows_v, grows_v, outv,
          n0_v, rcp_v, t0_v, gsem, tsem):
    c = lax.axis_index("c")
    s = lax.axis_index("s")
    wid = s * NC + c
    base = wid * BPW

    pltpu.sync_copy(tok_bmaj_h.at[wid], tokb_v)
    pltpu.sync_copy(tok_tmaj_h.at[wid], tokt_v)
    pltpu.sync_copy(title_idx_h.at[wid], tidx_v)

    title_descs = [
        pltpu.async_copy(title_table_h.at[tidx_v.at[j]],
                         trows_v.at[pl.ds(j * 128, 128)], tsem)
        for j in range(BPW // 128)
    ]

    pltpu.sync_copy(text_table_h.at[pl.ds(0, 8)], t0_v)
    t0a = t0_v[0, pl.ds(0, 16)]
    t0b = t0_v[0, pl.ds(16, 16)]

    @pl.loop(0, BPW // 16)
    def _cnt(g):
        cnt = jnp.zeros((16,), jnp.float32)
        for t in range(L):
            ids = tokt_v[t, pl.ds(g * 16, 16)]
            cnt = cnt + jnp.where(ids != 0, 1.0, 0.0).astype(jnp.float32)
        n0_v[pl.ds(g * 16, 16)] = jnp.float32(L) - cnt
        rcp_v[pl.ds(g * 16, 16)] = 1.0 / jnp.maximum(cnt, 1.0)

    @pl.loop(0, NCHUNK)
    def _chunk(g):
        descs = [
            pltpu.async_copy(
                text_table_h.at[tokb_v.at[g * IDX_ROWS_PER_CHUNK + j]],
                grows_v.at[pl.ds(j * 128, 128)], gsem)
            for j in range(IDX_ROWS_PER_CHUNK)
        ]
        for d in descs:
            d.wait()

        @pl.loop(0, CHUNK)
        def _acc(bl):
            r0 = bl * L
            acc0 = grows_v[r0, pl.ds(0, 16)]
            acc1 = grows_v[r0, pl.ds(16, 16)]
            for t in range(1, L):
                acc0 = acc0 + grows_v[r0 + t, pl.ds(0, 16)]
                acc1 = acc1 + grows_v[r0 + t, pl.ds(16, 16)]
            b = g * CHUNK + bl
            n0 = n0_v[pl.ds(b, 16)][0]
            rcp = rcp_v[pl.ds(b, 16)][0]
            outv[b, pl.ds(32, 16)] = (acc0 - n0 * t0a) * rcp
            outv[b, pl.ds(48, 16)] = (acc1 - n0 * t0b) * rcp

    for d in title_descs:
        d.wait()

    @pl.loop(0, BPW)
    def _title(b):
        outv[b, pl.ds(0, 16)] = trows_v[b, pl.ds(0, 16)]
        outv[b, pl.ds(16, 16)] = trows_v[b, pl.ds(16, 16)]

    pltpu.sync_copy(outv, out_h.at[pl.ds(base, BPW)])


@jax.jit
def kernel(title_ids, token_ids, title_table, text_table):
    title_idx = title_ids.reshape(NW, BPW // 128, 128)
    tok_bmaj = token_ids.reshape(NW, BPW * L // 128, 128)
    tok_tmaj = token_ids.T.reshape(L, NW, BPW).transpose(1, 0, 2)

    mesh = plsc.VectorSubcoreMesh(core_axis_name="c", subcore_axis_name="s")
    f = pl.kernel(
        _body,
        out_type=jax.ShapeDtypeStruct((B, 2 * D), jnp.float32),
        mesh=mesh,
        compiler_params=pltpu.CompilerParams(use_tc_tiling_on_sc=False),
        scratch_types=[
            pltpu.VMEM((BPW * L // 128, 128), jnp.int32),
            pltpu.VMEM((L, BPW), jnp.int32),
            pltpu.VMEM((BPW // 128, 128), jnp.int32),
            pltpu.VMEM((BPW, D), jnp.float32),
            pltpu.VMEM((ROWS_PER_CHUNK, D), jnp.float32),
            pltpu.VMEM((BPW, 2 * D), jnp.float32),
            pltpu.VMEM((BPW + 16,), jnp.float32),
            pltpu.VMEM((BPW + 16,), jnp.float32),
            pltpu.VMEM((8, D), jnp.float32),
            pltpu.SemaphoreType.DMA,
            pltpu.SemaphoreType.DMA,
        ],
    )
    return f(title_idx, tok_bmaj, tok_tmaj, title_table, text_table)


# speedup vs baseline: 14.8662x; 1.1825x over previous
import functools

import jax
import jax.numpy as jnp
from jax import lax
from jax.experimental import pallas as pl
from jax.experimental.pallas import tpu as pltpu
from jax.experimental.pallas import tpu_sc as plsc

B = 16384
L = 20
D = 32
TITLE_V = 100001
TEXT_V = 10000

NC = 2
NS = 16
NW = NC * NS
BPW = B // NW
CHUNK = 32
NCHUNK = BPW // CHUNK
ROWS_PER_CHUNK = CHUNK * L
IDX_ROWS_PER_CHUNK = ROWS_PER_CHUNK // 128


def _body(title_idx_h, tok_bmaj_h, title_table_h, text_table_h,
          out_h, tokb_v, tidx_v, trows_v, grows_v, outv,
          n0_v, rcp_v, t0_v, gsem0, gsem1, tsem):
    c = lax.axis_index("c")
    s = lax.axis_index("s")
    wid = s * NC + c
    base = wid * BPW

    pltpu.sync_copy(tok_bmaj_h.at[wid], tokb_v)
    pltpu.sync_copy(title_idx_h.at[wid], tidx_v)

    title_descs = [
        pltpu.async_copy(title_table_h.at[tidx_v.at[j]],
                         trows_v.at[pl.ds(j * 128, 128)], tsem)
        for j in range(BPW // 128)
    ]

    pltpu.sync_copy(text_table_h.at[pl.ds(0, 8)], t0_v)
    t0a = t0_v[0, pl.ds(0, 16)]
    t0b = t0_v[0, pl.ds(16, 16)]

    def issue(g, slot, sem):
        for j in range(IDX_ROWS_PER_CHUNK):
            pltpu.async_copy(
                text_table_h.at[tokb_v.at[g * IDX_ROWS_PER_CHUNK + j]],
                grows_v.at[slot, pl.ds(j * 128, 128)], sem)

    def drain(slot, sem):
        pltpu.make_async_copy(text_table_h.at[pl.ds(0, ROWS_PER_CHUNK)],
                              grows_v.at[slot], sem).wait()

    issue(0, 0, gsem0)
    issue(1, 1, gsem1)

    iota16 = lax.iota(jnp.int32, 16)

    @pl.loop(0, BPW // 16)
    def _cnt(g):
        f0 = (g * 16 + iota16) * L
        cnt = jnp.zeros((16,), jnp.float32)
        for t in range(L):
            f = f0 + t
            ids = plsc.load_gather(tokb_v, [f >> 7, f & 127])
            cnt = cnt + jnp.where(ids != 0, 1.0, 0.0).astype(jnp.float32)
        n0_v[pl.ds(g * 16, 16)] = jnp.float32(L) - cnt
        rcp_v[pl.ds(g * 16, 16)] = 1.0 / jnp.maximum(cnt, 1.0)

    def accumulate(g, slot):
        @pl.loop(0, CHUNK)
        def _acc(bl):
            r0 = bl * L
            acc0 = grows_v[slot, r0, pl.ds(0, 16)]
            acc1 = grows_v[slot, r0, pl.ds(16, 16)]
            for t in range(1, L):
                acc0 = acc0 + grows_v[slot, r0 + t, pl.ds(0, 16)]
                acc1 = acc1 + grows_v[slot, r0 + t, pl.ds(16, 16)]
            b = g * CHUNK + bl
            n0 = n0_v[pl.ds(b, 16)][0]
            rcp = rcp_v[pl.ds(b, 16)][0]
            outv[b, pl.ds(32, 16)] = (acc0 - n0 * t0a) * rcp
            outv[b, pl.ds(48, 16)] = (acc1 - n0 * t0b) * rcp

    @pl.loop(0, NCHUNK, step=2)
    def _chunk(g):
        drain(0, gsem0)

        @pl.when(g + 2 < NCHUNK)
        def _():
            issue(g + 2, 0, gsem0)

        accumulate(g, 0)
        drain(1, gsem1)

        @pl.when(g + 3 < NCHUNK)
        def _():
            issue(g + 3, 1, gsem1)

        accumulate(g + 1, 1)

    for d in title_descs:
        d.wait()

    @pl.loop(0, BPW)
    def _title(b):
        outv[b, pl.ds(0, 16)] = trows_v[b, pl.ds(0, 16)]
        outv[b, pl.ds(16, 16)] = trows_v[b, pl.ds(16, 16)]

    pltpu.sync_copy(outv, out_h.at[pl.ds(base, BPW)])


@jax.jit
def kernel(title_ids, token_ids, title_table, text_table):
    title_idx = title_ids.reshape(NW, BPW // 128, 128)
    tok_bmaj = token_ids.reshape(NW, BPW * L // 128, 128)

    mesh = plsc.VectorSubcoreMesh(core_axis_name="c", subcore_axis_name="s")
    f = pl.kernel(
        _body,
        out_type=jax.ShapeDtypeStruct((B, 2 * D), jnp.float32),
        mesh=mesh,
        compiler_params=pltpu.CompilerParams(use_tc_tiling_on_sc=False,
                                             needs_layout_passes=False),
        scratch_types=[
            pltpu.VMEM((BPW * L // 128, 128), jnp.int32),
            pltpu.VMEM((BPW // 128, 128), jnp.int32),
            pltpu.VMEM((BPW, D), jnp.float32),
            pltpu.VMEM((2, ROWS_PER_CHUNK, D), jnp.float32),
            pltpu.VMEM((BPW, 2 * D), jnp.float32),
            pltpu.VMEM((BPW + 16,), jnp.float32),
            pltpu.VMEM((BPW + 16,), jnp.float32),
            pltpu.VMEM((8, D), jnp.float32),
            pltpu.SemaphoreType.DMA,
            pltpu.SemaphoreType.DMA,
            pltpu.SemaphoreType.DMA,
        ],
    )
    return f(title_idx, tok_bmaj, title_table, text_table)


# speedup vs baseline: 17.0087x; 1.1441x over previous
import functools

import jax
import jax.numpy as jnp
from jax import lax
from jax.experimental import pallas as pl
from jax.experimental.pallas import tpu as pltpu
from jax.experimental.pallas import tpu_sc as plsc

B = 16384
L = 20
D = 32
TITLE_V = 100001
TEXT_V = 10000

NC = 2
NS = 16
NW = NC * NS
BPW = B // NW
CHUNK = 32
NCHUNK = BPW // CHUNK
ROWS_PER_CHUNK = CHUNK * L
IDX_ROWS_PER_CHUNK = ROWS_PER_CHUNK // 128
NIDXROWS = BPW * L // 128
TROWS = BPW * D // 128
TITLE_STREAMS_PER_CHUNK = TROWS // NCHUNK


def _body(title_idx_h, tokt_h, title_flat_h, text_table_h,
          out_h, tokt_v, tokb_v, tidx_v, teidx_v, trows_v, grows_v,
          outv, n0_v, rcp_v, t0_v, gsem0, gsem1, tsem):
    c = lax.axis_index("c")
    s = lax.axis_index("s")
    wid = s * NC + c
    base = wid * BPW

    pltpu.sync_copy(title_idx_h.at[wid], tidx_v.at[pl.ds(0, BPW)])
    pltpu.sync_copy(tokt_h.at[:, pl.ds(base, BPW)], tokt_v)

    pltpu.sync_copy(text_table_h.at[pl.ds(0, 8)], t0_v)
    t0a = t0_v[0, pl.ds(0, 16)]
    t0b = t0_v[0, pl.ds(16, 16)]

    iota16 = lax.iota(jnp.int32, 16)

    @pl.loop(0, BPW // 16)
    def _cnt(g):
        cnt = jnp.zeros((16,), jnp.float32)
        for t in range(L):
            ids = tokt_v[t, pl.ds(g * 16, 16)]
            cnt = cnt + jnp.where(ids != 0, 1.0, 0.0).astype(jnp.float32)
        n0_v[pl.ds(g * 16, 16)] = jnp.float32(L) - cnt
        rcp_v[pl.ds(g * 16, 16)] = 1.0 / jnp.maximum(cnt, 1.0)

    @pl.loop(0, NIDXROWS)
    def _mkidx(r):
        f0 = r * 128
        for k in range(8):
            f = f0 + k * 16 + iota16
            b = f // L
            t = f - b * L
            tokb_v[r, pl.ds(k * 16, 16)] = plsc.load_gather(tokt_v, [t, b])

    dvec0 = iota16 * TITLE_V
    dvec1 = (iota16 + 16) * TITLE_V

    @pl.loop(0, BPW)
    def _mktitle(bl):
        v = tidx_v[pl.ds(bl, 16)][0]
        r = bl >> 2
        col = (bl & 3) * D
        teidx_v[r, pl.ds(col, 16)] = dvec0 + v
        teidx_v[r, pl.ds(col + 16, 16)] = dvec1 + v

    def issue(g, slot, sem):
        for j in range(IDX_ROWS_PER_CHUNK):
            pltpu.async_copy(
                text_table_h.at[tokb_v.at[g * IDX_ROWS_PER_CHUNK + j]],
                grows_v.at[slot, pl.ds(j * 128, 128)], sem)

    def drain(slot, sem):
        pltpu.make_async_copy(text_table_h.at[pl.ds(0, ROWS_PER_CHUNK)],
                              grows_v.at[slot], sem).wait()

    issue(0, 0, gsem0)
    issue(1, 1, gsem1)

    def issue_title(g):
        for j in range(TITLE_STREAMS_PER_CHUNK):
            r = g * TITLE_STREAMS_PER_CHUNK + j
            pltpu.async_copy(title_flat_h.at[teidx_v.at[r]],
                             trows_v.at[pl.ds(r * 128, 128)], tsem)

    def accumulate(g, slot):
        @pl.loop(0, CHUNK)
        def _acc(bl):
            r0 = bl * L
            acc0 = grows_v[slot, r0, pl.ds(0, 16)]
            acc1 = grows_v[slot, r0, pl.ds(16, 16)]
            for t in range(1, L):
                acc0 = acc0 + grows_v[slot, r0 + t, pl.ds(0, 16)]
                acc1 = acc1 + grows_v[slot, r0 + t, pl.ds(16, 16)]
            b = g * CHUNK + bl
            n0 = n0_v[pl.ds(b, 16)][0]
            rcp = rcp_v[pl.ds(b, 16)][0]
            outv[b, pl.ds(32, 16)] = (acc0 - n0 * t0a) * rcp
            outv[b, pl.ds(48, 16)] = (acc1 - n0 * t0b) * rcp

    for g in range(NCHUNK):
        slot = g % 2
        sem = gsem0 if slot == 0 else gsem1
        drain(slot, sem)
        if g + 2 < NCHUNK:
            issue(g + 2, slot, sem)
        issue_title(g)
        accumulate(g, slot)

    pltpu.make_async_copy(title_flat_h.at[pl.ds(0, TROWS * 128)],
                          trows_v, tsem).wait()

    @pl.loop(0, BPW)
    def _title(bl):
        outv[bl, pl.ds(0, 16)] = trows_v[pl.ds(bl * D, 16)]
        outv[bl, pl.ds(16, 16)] = trows_v[pl.ds(bl * D + 16, 16)]

    pltpu.sync_copy(outv, out_h.at[pl.ds(base, BPW)])


@jax.jit
def kernel(title_ids, token_ids, title_table, text_table):
    title_idx = title_ids.reshape(NW, BPW)
    tokt = token_ids.T
    title_flat = title_table.T.reshape(-1)

    mesh = plsc.VectorSubcoreMesh(core_axis_name="c", subcore_axis_name="s")
    f = pl.kernel(
        _body,
        out_type=jax.ShapeDtypeStruct((B, 2 * D), jnp.float32),
        mesh=mesh,
        compiler_params=pltpu.CompilerParams(use_tc_tiling_on_sc=False,
                                             needs_layout_passes=False),
        scratch_types=[
            pltpu.VMEM((L, BPW), jnp.int32),
            pltpu.VMEM((NIDXROWS, 128), jnp.int32),
            pltpu.VMEM((BPW + 16,), jnp.int32),
            pltpu.VMEM((TROWS, 128), jnp.int32),
            pltpu.VMEM((TROWS * 128,), jnp.float32),
            pltpu.VMEM((2, ROWS_PER_CHUNK, D), jnp.float32),
            pltpu.VMEM((BPW, 2 * D), jnp.float32),
            pltpu.VMEM((BPW + 16,), jnp.float32),
            pltpu.VMEM((BPW + 16,), jnp.float32),
            pltpu.VMEM((8, D), jnp.float32),
            pltpu.SemaphoreType.DMA,
            pltpu.SemaphoreType.DMA,
            pltpu.SemaphoreType.DMA,
        ],
    )
    return f(title_idx, tokt, title_flat, text_table)
